# Initial kernel scaffold; baseline (speedup 1.0000x reference)
import jax
import jax.numpy as jnp
from jax.experimental import pallas as pl


def kernel(x, edge_index, W1a, b1a, W1b, b1b, W2a, b2a, W2b, b2b):
    raise NotImplementedError("write your pallas kernel here")


# speedup vs baseline: 32.8347x; 32.8347x over previous
import jax
import jax.numpy as jnp
from jax import lax
from jax.experimental import pallas as pl
from jax.experimental.pallas import tpu as pltpu
from jax.experimental.pallas import tpu_sc as plsc

N_NODES = 100000
N_PAD = 100096
ROWS_PER_TILE_N = N_PAD // 16
E_ROWS = 50176
K = 16


def _edge_loop(table_hbm, src_hbm, dst_hbm, shared, src_v, dst_v, rows_v,
               gsem, row_base, n_chunks):
  def chunk(g, carry):
    row0 = row_base + g * K
    pltpu.sync_copy(src_hbm.at[pl.ds(row0, K)], src_v)
    pltpu.sync_copy(dst_hbm.at[pl.ds(row0, K)], dst_v)
    descs = [pltpu.async_copy(table_hbm.at[src_v.at[j]], rows_v.at[j], gsem)
             for j in range(K)]
    for d in descs:
      d.wait()
    for j in range(K):
      pltpu.sync_copy(rows_v.at[j], shared.at[dst_v.at[j]], add=True)
    return carry

  lax.fori_loop(0, n_chunks, chunk, 0)


def _sc_agg_edge_split(table, src2d, dst2d, zeros):
  mesh = plsc.VectorSubcoreMesh(core_axis_name="c", subcore_axis_name="s")
  rows_per_tile = E_ROWS // 32

  def body(table_hbm, src_hbm, dst_hbm, zeros_hbm, out0, out1,
           src_v, dst_v, rows_v, shared, gsem):
    cid = lax.axis_index("c")
    sid = lax.axis_index("s")
    wid = cid * 16 + sid
    nbase = sid * ROWS_PER_TILE_N
    nslice = pl.ds(nbase, ROWS_PER_TILE_N)
    pltpu.sync_copy(zeros_hbm.at[nslice], shared.at[nslice])
    plsc.subcore_barrier()
    _edge_loop(table_hbm, src_hbm, dst_hbm, shared, src_v, dst_v, rows_v,
               gsem, wid * rows_per_tile, rows_per_tile // K)
    plsc.subcore_barrier()

    @pl.when(cid == 0)
    def _():
      pltpu.sync_copy(shared.at[nslice], out0.at[nslice])

    @pl.when(cid == 1)
    def _():
      pltpu.sync_copy(shared.at[nslice], out1.at[nslice])

  out_t = jax.ShapeDtypeStruct((N_PAD, 8), jnp.float32)
  return pl.kernel(
      body,
      out_type=(out_t, out_t),
      mesh=mesh,
      compiler_params=pltpu.CompilerParams(use_tc_tiling_on_sc=False),
      scratch_types=[
          pltpu.VMEM((K, 128), jnp.int32),
          pltpu.VMEM((K, 128), jnp.int32),
          pltpu.VMEM((K, 128, 8), jnp.float32),
          pltpu.VMEM_SHARED((N_PAD, 8), jnp.float32),
          pltpu.SemaphoreType.DMA,
      ],
  )(table, src2d, dst2d, zeros)


def _sc_agg_feat_split(tableL, tableR, src2d, dst2d, zeros):
  mesh = plsc.VectorSubcoreMesh(core_axis_name="c", subcore_axis_name="s")
  rows_per_tile = E_ROWS // 16

  def body(tl_hbm, tr_hbm, src_hbm, dst_hbm, zeros_hbm, outL, outR,
           src_v, dst_v, rows_v, shared, gsem):
    cid = lax.axis_index("c")
    sid = lax.axis_index("s")
    nbase = sid * ROWS_PER_TILE_N
    nslice = pl.ds(nbase, ROWS_PER_TILE_N)
    pltpu.sync_copy(zeros_hbm.at[nslice], shared.at[nslice])
    plsc.subcore_barrier()
    row_base = sid * rows_per_tile
    n_chunks = rows_per_tile // K

    @pl.when(cid == 0)
    def _():
      _edge_loop(tl_hbm, src_hbm, dst_hbm, shared, src_v, dst_v, rows_v,
                 gsem, row_base, n_chunks)

    @pl.when(cid == 1)
    def _():
      _edge_loop(tr_hbm, src_hbm, dst_hbm, shared, src_v, dst_v, rows_v,
                 gsem, row_base, n_chunks)

    plsc.subcore_barrier()

    @pl.when(cid == 0)
    def _():
      pltpu.sync_copy(shared.at[nslice], outL.at[nslice])

    @pl.when(cid == 1)
    def _():
      pltpu.sync_copy(shared.at[nslice], outR.at[nslice])

  out_t = jax.ShapeDtypeStruct((N_PAD, 8), jnp.float32)
  return pl.kernel(
      body,
      out_type=(out_t, out_t),
      mesh=mesh,
      compiler_params=pltpu.CompilerParams(use_tc_tiling_on_sc=False),
      scratch_types=[
          pltpu.VMEM((K, 128), jnp.int32),
          pltpu.VMEM((K, 128), jnp.int32),
          pltpu.VMEM((K, 128, 8), jnp.float32),
          pltpu.VMEM_SHARED((N_PAD, 8), jnp.float32),
          pltpu.SemaphoreType.DMA,
      ],
  )(tableL, tableR, src2d, dst2d, zeros)


def _tc_mlp1(xself, p0, p1, wa, ba, wb, bb):
  bm = 2048
  grid = (N_PAD + bm - 1) // bm

  def body(x_ref, p0_ref, p1_ref, wa_ref, ba_ref, wb_ref, bb_ref,
           oL_ref, oR_ref):
    h = x_ref[...] + p0_ref[...] + p1_ref[...]
    h = jnp.dot(h, wa_ref[...], preferred_element_type=jnp.float32)
    h = jnp.maximum(h + ba_ref[...], 0.0)
    o = jnp.dot(h, wb_ref[...], preferred_element_type=jnp.float32) + bb_ref[...]
    oL_ref[...] = o[:, :8]
    oR_ref[...] = o[:, 8:]

  node8 = pl.BlockSpec((bm, 8), lambda i: (i, 0))
  out_t = jax.ShapeDtypeStruct((N_PAD, 8), jnp.float32)
  return pl.pallas_call(
      body,
      grid=(grid,),
      in_specs=[
          node8, node8, node8,
          pl.BlockSpec((8, 16), lambda i: (0, 0)),
          pl.BlockSpec((1, 16), lambda i: (0, 0)),
          pl.BlockSpec((16, 16), lambda i: (0, 0)),
          pl.BlockSpec((1, 16), lambda i: (0, 0)),
      ],
      out_specs=(node8, node8),
      out_shape=(out_t, out_t),
  )(xself, p0, p1, wa, ba, wb, bb)


def _tc_mlp2(h1L, h1R, aggL, aggR, waL, waR, ba, wb, bb):
  bm = 2048
  grid = (N_PAD + bm - 1) // bm

  def body(hL_ref, hR_ref, aL_ref, aR_ref, waL_ref, waR_ref, ba_ref,
           wb_ref, bb_ref, o_ref):
    hl = hL_ref[...] + aL_ref[...]
    hr = hR_ref[...] + aR_ref[...]
    h = (jnp.dot(hl, waL_ref[...], preferred_element_type=jnp.float32)
         + jnp.dot(hr, waR_ref[...], preferred_element_type=jnp.float32))
    h = jnp.maximum(h + ba_ref[...], 0.0)
    o_ref[...] = (jnp.dot(h, wb_ref[...], preferred_element_type=jnp.float32)
                  + bb_ref[...])

  node8 = pl.BlockSpec((bm, 8), lambda i: (i, 0))
  return pl.pallas_call(
      body,
      grid=(grid,),
      in_specs=[
          node8, node8, node8, node8,
          pl.BlockSpec((8, 16), lambda i: (0, 0)),
          pl.BlockSpec((8, 16), lambda i: (0, 0)),
          pl.BlockSpec((1, 16), lambda i: (0, 0)),
          pl.BlockSpec((16, 2), lambda i: (0, 0)),
          pl.BlockSpec((1, 2), lambda i: (0, 0)),
      ],
      out_specs=pl.BlockSpec((bm, 2), lambda i: (i, 0)),
      out_shape=jax.ShapeDtypeStruct((N_PAD, 2), jnp.float32),
  )(h1L, h1R, aggL, aggR, waL, waR, ba, wb, bb)


@jax.jit
def kernel(x, edge_index, W1a, b1a, W1b, b1b, W2a, b2a, W2b, b2b):
  ei = edge_index.astype(jnp.int32)
  n_edges = ei.shape[1]
  pad_e = E_ROWS * 128 - n_edges
  src2d = jnp.concatenate(
      [ei[0], jnp.zeros((pad_e,), jnp.int32)]).reshape(E_ROWS, 128)
  dst2d = jnp.concatenate(
      [ei[1], jnp.full((pad_e,), N_NODES, jnp.int32)]).reshape(E_ROWS, 128)

  xp = jnp.zeros((N_PAD, 8), jnp.float32).at[:N_NODES, :5].set(x)
  zeros8 = jnp.zeros((N_PAD, 8), jnp.float32)
  W1a_p = jnp.zeros((8, 16), jnp.float32).at[:5, :].set(W1a)

  p0, p1 = _sc_agg_edge_split(xp, src2d, dst2d, zeros8)
  h1L, h1R = _tc_mlp1(xp, p0, p1, W1a_p, b1a.reshape(1, -1), W1b,
                      b1b.reshape(1, -1))
  aggL, aggR = _sc_agg_feat_split(h1L, h1R, src2d, dst2d, zeros8)
  out = _tc_mlp2(h1L, h1R, aggL, aggR, W2a[:8], W2a[8:],
                 b2a.reshape(1, -1), W2b, b2b.reshape(1, -1))
  return out[:N_NODES]


# speedup vs baseline: 41.5867x; 1.2665x over previous
import jax
import jax.numpy as jnp
from jax import lax
from jax.experimental import pallas as pl
from jax.experimental.pallas import tpu as pltpu
from jax.experimental.pallas import tpu_sc as plsc

N_NODES = 100000
N_PAD = 100096
ROWS_PER_TILE_N = N_PAD // 16
E_ROWS = 50176
K = 16


def _edge_loop(table_hbm, idx_hbm, shared, idx_v, rows_v, gsem, ssems,
               row_base, n_chunks):
  def stage_b(g, b):
    @pl.when((g >= 1) & (g <= n_chunks))
    def _():
      for j in range(K):
        pltpu.make_async_copy(table_hbm.at[idx_v.at[b, j, 0]],
                              rows_v.at[b, j], gsem).wait()
      for j in range(K):
        pltpu.async_copy(rows_v.at[b, j], shared.at[idx_v.at[b, j, 1]],
                         ssems[b], add=True)

  def stage_a(g, b):
    @pl.when((g >= 3) & (g <= n_chunks + 2))
    def _():
      for j in range(K):
        pltpu.make_async_copy(rows_v.at[b, j], shared.at[idx_v.at[b, j, 1]],
                              ssems[b]).wait()

    @pl.when(g <= n_chunks - 1)
    def _():
      pltpu.sync_copy(idx_hbm.at[pl.ds(row_base + g * K, K)], idx_v.at[b])
      for j in range(K):
        pltpu.async_copy(table_hbm.at[idx_v.at[b, j, 0]], rows_v.at[b, j],
                         gsem)

  def outer(go, carry):
    for p in range(3):
      g = go * 3 + p
      stage_b(g, (p + 2) % 3)
      stage_a(g, p)
    return carry

  n_outer = (n_chunks + 3 + 2) // 3
  lax.fori_loop(0, n_outer, outer, 0)


_SC_SCRATCH = [
    pltpu.VMEM((3, K, 2, 128), jnp.int32),
    pltpu.VMEM((3, K, 128, 8), jnp.float32),
    pltpu.VMEM_SHARED((N_PAD, 8), jnp.float32),
    pltpu.SemaphoreType.DMA,
    pltpu.SemaphoreType.DMA,
    pltpu.SemaphoreType.DMA,
    pltpu.SemaphoreType.DMA,
]


def _sc_agg_edge_split(table, idx2d, zeros):
  mesh = plsc.VectorSubcoreMesh(core_axis_name="c", subcore_axis_name="s")
  rows_per_tile = E_ROWS // 32

  def body(table_hbm, idx_hbm, zeros_hbm, out0, out1,
           idx_v, rows_v, shared, gsem, ssem0, ssem1, ssem2):
    cid = lax.axis_index("c")
    sid = lax.axis_index("s")
    wid = cid * 16 + sid
    nbase = sid * ROWS_PER_TILE_N
    nslice = pl.ds(nbase, ROWS_PER_TILE_N)
    pltpu.sync_copy(zeros_hbm.at[nslice], shared.at[nslice])
    plsc.subcore_barrier()
    _edge_loop(table_hbm, idx_hbm, shared, idx_v, rows_v, gsem,
               (ssem0, ssem1, ssem2), wid * rows_per_tile,
               rows_per_tile // K)
    plsc.subcore_barrier()

    @pl.when(cid == 0)
    def _():
      pltpu.sync_copy(shared.at[nslice], out0.at[nslice])

    @pl.when(cid == 1)
    def _():
      pltpu.sync_copy(shared.at[nslice], out1.at[nslice])

  out_t = jax.ShapeDtypeStruct((N_PAD, 8), jnp.float32)
  return pl.kernel(
      body,
      out_type=(out_t, out_t),
      mesh=mesh,
      compiler_params=pltpu.CompilerParams(use_tc_tiling_on_sc=False),
      scratch_types=_SC_SCRATCH,
  )(table, idx2d, zeros)


def _sc_agg_feat_split(tableL, tableR, idx2d, zeros):
  mesh = plsc.VectorSubcoreMesh(core_axis_name="c", subcore_axis_name="s")
  rows_per_tile = E_ROWS // 16

  def body(tl_hbm, tr_hbm, idx_hbm, zeros_hbm, outL, outR,
           idx_v, rows_v, shared, gsem, ssem0, ssem1, ssem2):
    cid = lax.axis_index("c")
    sid = lax.axis_index("s")
    nbase = sid * ROWS_PER_TILE_N
    nslice = pl.ds(nbase, ROWS_PER_TILE_N)
    pltpu.sync_copy(zeros_hbm.at[nslice], shared.at[nslice])
    plsc.subcore_barrier()
    row_base = sid * rows_per_tile
    n_chunks = rows_per_tile // K
    ssems = (ssem0, ssem1, ssem2)

    @pl.when(cid == 0)
    def _():
      _edge_loop(tl_hbm, idx_hbm, shared, idx_v, rows_v, gsem, ssems,
                 row_base, n_chunks)

    @pl.when(cid == 1)
    def _():
      _edge_loop(tr_hbm, idx_hbm, shared, idx_v, rows_v, gsem, ssems,
                 row_base, n_chunks)

    plsc.subcore_barrier()

    @pl.when(cid == 0)
    def _():
      pltpu.sync_copy(shared.at[nslice], outL.at[nslice])

    @pl.when(cid == 1)
    def _():
      pltpu.sync_copy(shared.at[nslice], outR.at[nslice])

  out_t = jax.ShapeDtypeStruct((N_PAD, 8), jnp.float32)
  return pl.kernel(
      body,
      out_type=(out_t, out_t),
      mesh=mesh,
      compiler_params=pltpu.CompilerParams(use_tc_tiling_on_sc=False),
      scratch_types=_SC_SCRATCH,
  )(tableL, tableR, idx2d, zeros)


def _tc_mlp1(xself, p0, p1, wa, ba, wb, bb):
  bm = 2048
  grid = (N_PAD + bm - 1) // bm

  def body(x_ref, p0_ref, p1_ref, wa_ref, ba_ref, wb_ref, bb_ref,
           oL_ref, oR_ref):
    h = x_ref[...] + p0_ref[...] + p1_ref[...]
    h = jnp.dot(h, wa_ref[...], preferred_element_type=jnp.float32)
    h = jnp.maximum(h + ba_ref[...], 0.0)
    o = jnp.dot(h, wb_ref[...], preferred_element_type=jnp.float32) + bb_ref[...]
    oL_ref[...] = o[:, :8]
    oR_ref[...] = o[:, 8:]

  node8 = pl.BlockSpec((bm, 8), lambda i: (i, 0))
  out_t = jax.ShapeDtypeStruct((N_PAD, 8), jnp.float32)
  return pl.pallas_call(
      body,
      grid=(grid,),
      in_specs=[
          node8, node8, node8,
          pl.BlockSpec((8, 16), lambda i: (0, 0)),
          pl.BlockSpec((1, 16), lambda i: (0, 0)),
          pl.BlockSpec((16, 16), lambda i: (0, 0)),
          pl.BlockSpec((1, 16), lambda i: (0, 0)),
      ],
      out_specs=(node8, node8),
      out_shape=(out_t, out_t),
  )(xself, p0, p1, wa, ba, wb, bb)


def _tc_mlp2(h1L, h1R, aggL, aggR, waL, waR, ba, wb, bb):
  bm = 2048
  grid = (N_PAD + bm - 1) // bm

  def body(hL_ref, hR_ref, aL_ref, aR_ref, waL_ref, waR_ref, ba_ref,
           wb_ref, bb_ref, o_ref):
    hl = hL_ref[...] + aL_ref[...]
    hr = hR_ref[...] + aR_ref[...]
    h = (jnp.dot(hl, waL_ref[...], preferred_element_type=jnp.float32)
         + jnp.dot(hr, waR_ref[...], preferred_element_type=jnp.float32))
    h = jnp.maximum(h + ba_ref[...], 0.0)
    o_ref[...] = (jnp.dot(h, wb_ref[...], preferred_element_type=jnp.float32)
                  + bb_ref[...])

  node8 = pl.BlockSpec((bm, 8), lambda i: (i, 0))
  return pl.pallas_call(
      body,
      grid=(grid,),
      in_specs=[
          node8, node8, node8, node8,
          pl.BlockSpec((8, 16), lambda i: (0, 0)),
          pl.BlockSpec((8, 16), lambda i: (0, 0)),
          pl.BlockSpec((1, 16), lambda i: (0, 0)),
          pl.BlockSpec((16, 2), lambda i: (0, 0)),
          pl.BlockSpec((1, 2), lambda i: (0, 0)),
      ],
      out_specs=pl.BlockSpec((bm, 2), lambda i: (i, 0)),
      out_shape=jax.ShapeDtypeStruct((N_PAD, 2), jnp.float32),
  )(h1L, h1R, aggL, aggR, waL, waR, ba, wb, bb)


@jax.jit
def kernel(x, edge_index, W1a, b1a, W1b, b1b, W2a, b2a, W2b, b2b):
  ei = edge_index.astype(jnp.int32)
  n_edges = ei.shape[1]
  pad_e = E_ROWS * 128 - n_edges
  src2d = jnp.concatenate(
      [ei[0], jnp.zeros((pad_e,), jnp.int32)]).reshape(E_ROWS, 1, 128)
  dst2d = jnp.concatenate(
      [ei[1], jnp.full((pad_e,), N_NODES, jnp.int32)]).reshape(E_ROWS, 1, 128)
  idx2d = jnp.concatenate([src2d, dst2d], axis=1)

  xp = jnp.zeros((N_PAD, 8), jnp.float32).at[:N_NODES, :5].set(x)
  zeros8 = jnp.zeros((N_PAD, 8), jnp.float32)
  W1a_p = jnp.zeros((8, 16), jnp.float32).at[:5, :].set(W1a)

  p0, p1 = _sc_agg_edge_split(xp, idx2d, zeros8)
  h1L, h1R = _tc_mlp1(xp, p0, p1, W1a_p, b1a.reshape(1, -1), W1b,
                      b1b.reshape(1, -1))
  aggL, aggR = _sc_agg_feat_split(h1L, h1R, idx2d, zeros8)
  out = _tc_mlp2(h1L, h1R, aggL, aggR, W2a[:8], W2a[8:],
                 b2a.reshape(1, -1), W2b, b2b.reshape(1, -1))
  return out[:N_NODES]
